# baseline (speedup 1.0000x reference)
import jax
import jax.numpy as jnp
from jax import lax
from jax.experimental import pallas as pl
from jax.experimental.pallas import tpu as pltpu
from jax.experimental.pallas import tpu_sc as plsc

N_NODES = 10000
N_EDGES = 160000
D_FEAT = 256
D_HALF = D_FEAT // 2

NUM_TILES = 16
CHUNK = 96
CHUNKS_PER_TILE = 105
E_PAD = NUM_TILES * CHUNKS_PER_TILE * CHUNK
N_DUMMY = 40
STRIPE = 632
LAST_STRIPE = N_NODES - (NUM_TILES - 1) * STRIPE
NBUF = 4
ROUNDS = 13
LAST = CHUNKS_PER_TILE - 1


def _clear_stripe(acc, rows0, base, nrows):
    nfull = nrows // CHUNK
    rest = nrows - nfull * CHUNK

    @pl.loop(0, nfull)
    def _clear(k):
        pltpu.sync_copy(rows0, acc.at[pl.ds(base + k * CHUNK, CHUNK)])

    pltpu.sync_copy(rows0.at[pl.ds(0, rest)],
                    acc.at[pl.ds(base + nfull * CHUNK, rest)])


def _sc_body(x_hbm, src_hbm, dst_hbm, out_hbm, *refs):
    rows = refs[0:4]
    stages = refs[4:12]
    acc = refs[12]
    semr = refs[13:17]
    semi = refs[17:25]
    c = lax.axis_index("c")
    s = lax.axis_index("s")

    @pl.loop(0, CHUNK)
    def _zero_rows(r):
        @pl.loop(0, D_HALF // 16)
        def _zero_lanes(j):
            rows[0][r, pl.ds(j * 16, 16)] = jnp.zeros((16,), jnp.float32)

    @pl.when(s < NUM_TILES - 1)
    def _clear_full():
        _clear_stripe(acc, rows[0], s * STRIPE, STRIPE)

    @pl.when(s == NUM_TILES - 1)
    def _clear_last():
        _clear_stripe(acc, rows[0], (NUM_TILES - 1) * STRIPE, LAST_STRIPE)

    plsc.subcore_barrier()

    def idx_issue(j, st):
        pltpu.async_copy(src_hbm.at[s].at[j], stages[st].at[pl.ds(0, 1)],
                         semi[st])
        pltpu.async_copy(dst_hbm.at[s].at[j], stages[st].at[pl.ds(1, 1)],
                         semi[st])

    def idx_wait(j, st):
        pltpu.make_async_copy(src_hbm.at[s].at[j],
                              stages[st].at[pl.ds(0, 1)], semi[st]).wait()
        pltpu.make_async_copy(dst_hbm.at[s].at[j],
                              stages[st].at[pl.ds(1, 1)], semi[st]).wait()

    def gather(j, b, st):
        return pltpu.async_copy(x_hbm.at[c].at[stages[st].at[0]],
                                rows[b], semr[b])

    def wait_gather(j, b, st):
        pltpu.make_async_copy(x_hbm.at[c].at[stages[st].at[0]],
                              rows[b], semr[b]).wait()

    def scatter(b, st):
        return pltpu.async_copy(rows[b], acc.at[stages[st].at[1]],
                                semr[b], add=True)

    for j in range(8):
        idx_issue(j, (j % 4) * 2 + j // 4)
    for j in range(4):
        idx_wait(j, j * 2)
        gather(j, j, j * 2)

    sc = [None] * NBUF

    @pl.loop(0, ROUNDS)
    def _round(i):
        j8 = 8 * i
        for b in range(4):
            wait_gather(j8 + b, b, b * 2)
            sc[b] = scatter(b, b * 2)
        for b in range(4):
            sc[b].wait()
            idx_issue(jnp.minimum(j8 + 8 + b, LAST), b * 2)
            idx_wait(j8 + 4 + b, b * 2 + 1)
            gather(j8 + 4 + b, b, b * 2 + 1)
        for b in range(4):
            wait_gather(j8 + 4 + b, b, b * 2 + 1)
            sc[b] = scatter(b, b * 2 + 1)
        for b in range(4):
            sc[b].wait()
            idx_issue(jnp.minimum(j8 + 12 + b, LAST), b * 2 + 1)
            idx_wait(jnp.minimum(j8 + 8 + b, LAST), b * 2)
            gather(jnp.minimum(j8 + 8 + b, LAST), b, b * 2)

    wait_gather(LAST, 0, 0)
    pltpu.sync_copy(rows[0], acc.at[stages[0].at[1]], add=True)
    for b in range(1, 4):
        wait_gather(LAST, b, b * 2)
    for b in range(4):
        idx_wait(LAST, b * 2 + 1)

    plsc.subcore_barrier()

    @pl.when(s < NUM_TILES - 1)
    def _store_full():
        pltpu.sync_copy(acc.at[pl.ds(s * STRIPE, STRIPE)],
                        out_hbm.at[c].at[pl.ds(s * STRIPE, STRIPE)])

    @pl.when(s == NUM_TILES - 1)
    def _store_last():
        pltpu.sync_copy(
            acc.at[pl.ds((NUM_TILES - 1) * STRIPE, LAST_STRIPE)],
            out_hbm.at[c].at[pl.ds((NUM_TILES - 1) * STRIPE, LAST_STRIPE)])


@jax.jit
def _message_passing(x2, src_r, dst_r):
    mesh = plsc.VectorSubcoreMesh(core_axis_name="c", subcore_axis_name="s")
    run = pl.kernel(
        _sc_body,
        out_type=jax.ShapeDtypeStruct((2, N_NODES, D_HALF), jnp.float32),
        mesh=mesh,
        scratch_types=(
            [pltpu.VMEM((CHUNK, D_HALF), jnp.float32)] * NBUF +
            [pltpu.VMEM((2, CHUNK), jnp.int32)] * 8 +
            [pltpu.VMEM_SHARED((N_NODES + N_DUMMY, D_HALF),
                               jnp.float32)] +
            [pltpu.SemaphoreType.DMA] * 12
        ),
        compiler_params=pltpu.CompilerParams(use_tc_tiling_on_sc=False),
    )
    return run(x2, src_r, dst_r)


def _chunked_indices(idx, pad):
    padded = jnp.concatenate([idx, pad])
    return padded.reshape(NUM_TILES, CHUNKS_PER_TILE, 1, CHUNK)


def kernel(x, edge_index):
    x2 = jnp.moveaxis(x.reshape(N_NODES, 2, D_HALF), 1, 0)
    n_pad = E_PAD - N_EDGES
    src_r = _chunked_indices(edge_index[0],
                             jnp.zeros((n_pad,), jnp.int32))
    dst_r = _chunked_indices(edge_index[1],
                             N_NODES + jnp.arange(n_pad, dtype=jnp.int32)
                             % N_DUMMY)
    out2 = _message_passing(x2, src_r, dst_r)
    return jnp.moveaxis(out2, 0, 1).reshape(N_NODES, D_FEAT)


# speedup vs baseline: 1.2288x; 1.2288x over previous
import jax
import jax.numpy as jnp
from jax import lax
from jax.experimental import pallas as pl
from jax.experimental.pallas import tpu as pltpu
from jax.experimental.pallas import tpu_sc as plsc

N_NODES = 10000
N_EDGES = 160000
D_FEAT = 256
D_HALF = D_FEAT // 2

NUM_TILES = 16
CHUNK = 64
CHUNKS_PER_TILE = 157
E_PAD = NUM_TILES * CHUNKS_PER_TILE * CHUNK
N_DUMMY = 40
STRIPE = 632
LAST_STRIPE = N_NODES - (NUM_TILES - 1) * STRIPE
NBUF = 4
ROUNDS = 19
TAIL_BASE = 8 * ROUNDS
LAST = CHUNKS_PER_TILE - 1


def _clear_stripe(acc, rows0, base, nrows):
    nfull = nrows // CHUNK
    rest = nrows - nfull * CHUNK

    @pl.loop(0, nfull)
    def _clear(k):
        pltpu.sync_copy(rows0, acc.at[pl.ds(base + k * CHUNK, CHUNK)])

    pltpu.sync_copy(rows0.at[pl.ds(0, rest)],
                    acc.at[pl.ds(base + nfull * CHUNK, rest)])


def _sc_body(x_hbm, src_hbm, dst_hbm, out_hbm, *refs):
    rows = refs[0:4]
    stages = refs[4:12]
    acc = refs[12]
    semr = refs[13:17]
    semi = refs[17:25]
    c = lax.axis_index("c")
    s = lax.axis_index("s")

    @pl.loop(0, CHUNK)
    def _zero_rows(r):
        @pl.loop(0, D_HALF // 16)
        def _zero_lanes(j):
            rows[0][r, pl.ds(j * 16, 16)] = jnp.zeros((16,), jnp.float32)

    @pl.when(s < NUM_TILES - 1)
    def _clear_full():
        _clear_stripe(acc, rows[0], s * STRIPE, STRIPE)

    @pl.when(s == NUM_TILES - 1)
    def _clear_last():
        _clear_stripe(acc, rows[0], (NUM_TILES - 1) * STRIPE, LAST_STRIPE)

    plsc.subcore_barrier()

    def idx_issue(j, st):
        pltpu.async_copy(src_hbm.at[s].at[j], stages[st].at[pl.ds(0, 1)],
                         semi[st])
        pltpu.async_copy(dst_hbm.at[s].at[j], stages[st].at[pl.ds(1, 1)],
                         semi[st])

    def idx_wait(j, st):
        pltpu.make_async_copy(src_hbm.at[s].at[j],
                              stages[st].at[pl.ds(0, 1)], semi[st]).wait()
        pltpu.make_async_copy(dst_hbm.at[s].at[j],
                              stages[st].at[pl.ds(1, 1)], semi[st]).wait()

    def gather(j, b, st):
        return pltpu.async_copy(x_hbm.at[c].at[stages[st].at[0]],
                                rows[b], semr[b])

    def wait_gather(j, b, st):
        pltpu.make_async_copy(x_hbm.at[c].at[stages[st].at[0]],
                              rows[b], semr[b]).wait()

    def scatter(b, st):
        return pltpu.async_copy(rows[b], acc.at[stages[st].at[1]],
                                semr[b], add=True)

    for j in range(8):
        idx_issue(j, (j % 4) * 2 + j // 4)
    for j in range(4):
        idx_wait(j, j * 2)
        gather(j, j, j * 2)

    sc = [None] * NBUF

    @pl.loop(0, ROUNDS)
    def _round(i):
        j8 = 8 * i
        for b in range(4):
            wait_gather(j8 + b, b, b * 2)
            sc[b] = scatter(b, b * 2)
        for b in range(4):
            sc[b].wait()
            idx_issue(jnp.minimum(j8 + 8 + b, LAST), b * 2)
            idx_wait(j8 + 4 + b, b * 2 + 1)
            gather(j8 + 4 + b, b, b * 2 + 1)
        for b in range(4):
            wait_gather(j8 + 4 + b, b, b * 2 + 1)
            sc[b] = scatter(b, b * 2 + 1)
        for b in range(4):
            sc[b].wait()
            idx_issue(jnp.minimum(j8 + 12 + b, LAST), b * 2 + 1)
            idx_wait(jnp.minimum(j8 + 8 + b, LAST), b * 2)
            gather(jnp.minimum(j8 + 8 + b, LAST), b, b * 2)

    for b in range(4):
        wait_gather(TAIL_BASE + b, b, b * 2)
        sc[b] = scatter(b, b * 2)
    sc[0].wait()
    idx_wait(LAST, 1)
    g = gather(LAST, 0, 1)
    for b in range(1, 4):
        sc[b].wait()
        idx_wait(LAST, b * 2 + 1)
    g.wait()
    pltpu.sync_copy(rows[0], acc.at[stages[1].at[1]], add=True)

    plsc.subcore_barrier()

    @pl.when(s < NUM_TILES - 1)
    def _store_full():
        pltpu.sync_copy(acc.at[pl.ds(s * STRIPE, STRIPE)],
                        out_hbm.at[c].at[pl.ds(s * STRIPE, STRIPE)])

    @pl.when(s == NUM_TILES - 1)
    def _store_last():
        pltpu.sync_copy(
            acc.at[pl.ds((NUM_TILES - 1) * STRIPE, LAST_STRIPE)],
            out_hbm.at[c].at[pl.ds((NUM_TILES - 1) * STRIPE, LAST_STRIPE)])


@jax.jit
def _message_passing(x2, src_r, dst_r):
    mesh = plsc.VectorSubcoreMesh(core_axis_name="c", subcore_axis_name="s")
    run = pl.kernel(
        _sc_body,
        out_type=jax.ShapeDtypeStruct((2, N_NODES, D_HALF), jnp.float32),
        mesh=mesh,
        scratch_types=(
            [pltpu.VMEM((CHUNK, D_HALF), jnp.float32)] * NBUF +
            [pltpu.VMEM((2, CHUNK), jnp.int32)] * 8 +
            [pltpu.VMEM_SHARED((N_NODES + N_DUMMY, D_HALF),
                               jnp.float32)] +
            [pltpu.SemaphoreType.DMA] * 12
        ),
        compiler_params=pltpu.CompilerParams(use_tc_tiling_on_sc=False),
    )
    return run(x2, src_r, dst_r)


def _chunked_indices(idx, pad):
    padded = jnp.concatenate([idx, pad])
    return padded.reshape(NUM_TILES, CHUNKS_PER_TILE, 1, CHUNK)


def kernel(x, edge_index):
    x2 = jnp.moveaxis(x.reshape(N_NODES, 2, D_HALF), 1, 0)
    n_pad = E_PAD - N_EDGES
    src_r = _chunked_indices(edge_index[0],
                             jnp.zeros((n_pad,), jnp.int32))
    dst_r = _chunked_indices(edge_index[1],
                             N_NODES + jnp.arange(n_pad, dtype=jnp.int32)
                             % N_DUMMY)
    out2 = _message_passing(x2, src_r, dst_r)
    return jnp.moveaxis(out2, 0, 1).reshape(N_NODES, D_FEAT)


# speedup vs baseline: 1.4964x; 1.2177x over previous
import jax
import jax.numpy as jnp
from jax import lax
from jax.experimental import pallas as pl
from jax.experimental.pallas import tpu as pltpu
from jax.experimental.pallas import tpu_sc as plsc

N_NODES = 10000
N_EDGES = 160000
D_FEAT = 256
D_HALF = D_FEAT // 2

NUM_TILES = 16
CHUNK = 80
CHUNKS_TOTAL = N_EDGES // CHUNK
CHUNKS_PER_TILE = CHUNKS_TOTAL // NUM_TILES
STRIPE = 632
LAST_STRIPE = N_NODES - (NUM_TILES - 1) * STRIPE
NBUF = 4
ROUNDS = 15
LAST = CHUNKS_PER_TILE - 1


def _clear_stripe(acc, rows0, base, nrows):
    nfull = nrows // CHUNK
    rest = nrows - nfull * CHUNK

    @pl.loop(0, nfull)
    def _clear(k):
        pltpu.sync_copy(rows0, acc.at[pl.ds(base + k * CHUNK, CHUNK)])

    pltpu.sync_copy(rows0.at[pl.ds(0, rest)],
                    acc.at[pl.ds(base + nfull * CHUNK, rest)])


def _sc_body(x_hbm, sd_hbm, out_hbm, *refs):
    rows = refs[0:4]
    stages = refs[4:12]
    acc = refs[12]
    semr = refs[13:17]
    semi = refs[17:25]
    c = lax.axis_index("c")
    s = lax.axis_index("s")

    @pl.loop(0, CHUNK)
    def _zero_rows(r):
        @pl.loop(0, D_HALF // 16)
        def _zero_lanes(j):
            rows[0][r, pl.ds(j * 16, 16)] = jnp.zeros((16,), jnp.float32)

    @pl.when(s < NUM_TILES - 1)
    def _clear_full():
        _clear_stripe(acc, rows[0], s * STRIPE, STRIPE)

    @pl.when(s == NUM_TILES - 1)
    def _clear_last():
        _clear_stripe(acc, rows[0], (NUM_TILES - 1) * STRIPE, LAST_STRIPE)

    plsc.subcore_barrier()

    def idx_issue(j, st):
        pltpu.async_copy(sd_hbm.at[s].at[j], stages[st], semi[st])

    def idx_wait(j, st):
        pltpu.make_async_copy(sd_hbm.at[s].at[j], stages[st],
                              semi[st]).wait()

    def gather(j, b, st):
        return pltpu.async_copy(x_hbm.at[c].at[stages[st].at[0]],
                                rows[b], semr[b])

    def wait_gather(j, b, st):
        pltpu.make_async_copy(x_hbm.at[c].at[stages[st].at[0]],
                              rows[b], semr[b]).wait()

    def scatter(b, st):
        return pltpu.async_copy(rows[b], acc.at[stages[st].at[1]],
                                semr[b], add=True)

    for j in range(8):
        idx_issue(j, (j % 4) * 2 + j // 4)
    for j in range(4):
        idx_wait(j, j * 2)
        gather(j, j, j * 2)

    sc = [None] * NBUF

    @pl.loop(0, ROUNDS)
    def _round(i):
        j8 = 8 * i
        for b in range(4):
            wait_gather(j8 + b, b, b * 2)
            sc[b] = scatter(b, b * 2)
        for b in range(4):
            sc[b].wait()
            idx_issue(jnp.minimum(j8 + 8 + b, LAST), b * 2)
            idx_wait(j8 + 4 + b, b * 2 + 1)
            gather(j8 + 4 + b, b, b * 2 + 1)
        for b in range(4):
            wait_gather(j8 + 4 + b, b, b * 2 + 1)
            sc[b] = scatter(b, b * 2 + 1)
        for b in range(4):
            sc[b].wait()
            idx_issue(jnp.minimum(j8 + 12 + b, LAST), b * 2 + 1)
            idx_wait(jnp.minimum(j8 + 8 + b, LAST), b * 2)
            gather(jnp.minimum(j8 + 8 + b, LAST), b, b * 2)

    for b in range(4):
        wait_gather(120 + b, b, b * 2)
        sc[b] = scatter(b, b * 2)
    sc[0].wait()
    idx_wait(LAST, 1)
    g = gather(LAST, 0, 1)
    for b in range(1, 4):
        sc[b].wait()
        idx_wait(LAST, b * 2 + 1)
    g.wait()
    pltpu.sync_copy(rows[0], acc.at[stages[1].at[1]], add=True)

    plsc.subcore_barrier()

    @pl.when(s < NUM_TILES - 1)
    def _store_full():
        pltpu.sync_copy(acc.at[pl.ds(s * STRIPE, STRIPE)],
                        out_hbm.at[c].at[pl.ds(s * STRIPE, STRIPE)])

    @pl.when(s == NUM_TILES - 1)
    def _store_last():
        pltpu.sync_copy(
            acc.at[pl.ds((NUM_TILES - 1) * STRIPE, LAST_STRIPE)],
            out_hbm.at[c].at[pl.ds((NUM_TILES - 1) * STRIPE, LAST_STRIPE)])


@jax.jit
def _message_passing(x2, sd_r):
    mesh = plsc.VectorSubcoreMesh(core_axis_name="c", subcore_axis_name="s")
    run = pl.kernel(
        _sc_body,
        out_type=jax.ShapeDtypeStruct((2, N_NODES, D_HALF), jnp.float32),
        mesh=mesh,
        scratch_types=(
            [pltpu.VMEM((CHUNK, D_HALF), jnp.float32)] * NBUF +
            [pltpu.VMEM((2, CHUNK), jnp.int32)] * 8 +
            [pltpu.VMEM_SHARED((N_NODES, D_HALF), jnp.float32)] +
            [pltpu.SemaphoreType.DMA] * 12
        ),
        compiler_params=pltpu.CompilerParams(use_tc_tiling_on_sc=False),
    )
    return run(x2, sd_r)


def kernel(x, edge_index):
    x2 = jnp.moveaxis(x.reshape(N_NODES, 2, D_HALF), 1, 0)
    sd_r = jnp.stack(
        [edge_index[0].reshape(NUM_TILES, CHUNKS_PER_TILE, CHUNK),
         edge_index[1].reshape(NUM_TILES, CHUNKS_PER_TILE, CHUNK)],
        axis=2)
    out2 = _message_passing(x2, sd_r)
    return jnp.moveaxis(out2, 0, 1).reshape(N_NODES, D_FEAT)


# speedup vs baseline: 1.5349x; 1.0257x over previous
import jax
import jax.numpy as jnp
from jax import lax
from jax.experimental import pallas as pl
from jax.experimental.pallas import tpu as pltpu
from jax.experimental.pallas import tpu_sc as plsc

N_NODES = 10000
N_EDGES = 160000
D_FEAT = 256
D_HALF = D_FEAT // 2

NUM_TILES = 16
CHUNK = 80
CHUNKS_TOTAL = N_EDGES // CHUNK
CHUNKS_PER_TILE = CHUNKS_TOTAL // NUM_TILES
STRIPE = 632
LAST_STRIPE = N_NODES - (NUM_TILES - 1) * STRIPE
NBUF = 4
ROUNDS = 15
LAST = CHUNKS_PER_TILE - 1


def _clear_stripe(acc, rows0, base, nrows):
    nfull = nrows // CHUNK
    rest = nrows - nfull * CHUNK

    @pl.loop(0, nfull)
    def _clear(k):
        pltpu.sync_copy(rows0, acc.at[pl.ds(base + k * CHUNK, CHUNK)])

    pltpu.sync_copy(rows0.at[pl.ds(0, rest)],
                    acc.at[pl.ds(base + nfull * CHUNK, rest)])


def _sc_body(x_hbm, src_hbm, dst_hbm, out_hbm, *refs):
    rows = refs[0:4]
    stages = refs[4:12]
    acc = refs[12]
    semr = refs[13:17]
    semi = refs[17:25]
    c = lax.axis_index("c")
    s = lax.axis_index("s")

    def idx_issue(j, st):
        pltpu.async_copy(src_hbm.at[s].at[j], stages[st].at[pl.ds(0, 1)],
                         semi[st])
        pltpu.async_copy(dst_hbm.at[s].at[j], stages[st].at[pl.ds(1, 1)],
                         semi[st])

    def idx_wait(j, st):
        pltpu.make_async_copy(src_hbm.at[s].at[j],
                              stages[st].at[pl.ds(0, 1)], semi[st]).wait()
        pltpu.make_async_copy(dst_hbm.at[s].at[j],
                              stages[st].at[pl.ds(1, 1)], semi[st]).wait()

    def gather(j, b, st):
        return pltpu.async_copy(x_hbm.at[c].at[stages[st].at[0]],
                                rows[b], semr[b])

    def wait_gather(j, b, st):
        pltpu.make_async_copy(x_hbm.at[c].at[stages[st].at[0]],
                              rows[b], semr[b]).wait()

    def scatter(b, st):
        return pltpu.async_copy(rows[b], acc.at[stages[st].at[1]],
                                semr[b], add=True)

    for j in range(8):
        idx_issue(j, (j % 4) * 2 + j // 4)
    for j in range(1, 4):
        idx_wait(j, j * 2)
        gather(j, j, j * 2)

    @pl.loop(0, CHUNK)
    def _zero_rows(r):
        @pl.loop(0, D_HALF // 16)
        def _zero_lanes(j):
            rows[0][r, pl.ds(j * 16, 16)] = jnp.zeros((16,), jnp.float32)

    @pl.when(s < NUM_TILES - 1)
    def _clear_full():
        _clear_stripe(acc, rows[0], s * STRIPE, STRIPE)

    @pl.when(s == NUM_TILES - 1)
    def _clear_last():
        _clear_stripe(acc, rows[0], (NUM_TILES - 1) * STRIPE, LAST_STRIPE)

    idx_wait(0, 0)
    gather(0, 0, 0)

    plsc.subcore_barrier()

    sc = [None] * NBUF

    @pl.loop(0, ROUNDS)
    def _round(i):
        j8 = 8 * i
        for b in range(4):
            wait_gather(j8 + b, b, b * 2)
            sc[b] = scatter(b, b * 2)
        for b in range(4):
            sc[b].wait()
            idx_issue(jnp.minimum(j8 + 8 + b, LAST), b * 2)
            idx_wait(j8 + 4 + b, b * 2 + 1)
            gather(j8 + 4 + b, b, b * 2 + 1)
        for b in range(4):
            wait_gather(j8 + 4 + b, b, b * 2 + 1)
            sc[b] = scatter(b, b * 2 + 1)
        for b in range(4):
            sc[b].wait()
            idx_issue(jnp.minimum(j8 + 12 + b, LAST), b * 2 + 1)
            idx_wait(jnp.minimum(j8 + 8 + b, LAST), b * 2)
            gather(jnp.minimum(j8 + 8 + b, LAST), b, b * 2)

    for b in range(4):
        wait_gather(120 + b, b, b * 2)
        sc[b] = scatter(b, b * 2)
    sc[0].wait()
    idx_wait(LAST, 1)
    g = gather(LAST, 0, 1)
    for b in range(1, 4):
        sc[b].wait()
        idx_wait(LAST, b * 2 + 1)
    g.wait()
    pltpu.sync_copy(rows[0], acc.at[stages[1].at[1]], add=True)

    plsc.subcore_barrier()

    @pl.when(s < NUM_TILES - 1)
    def _store_full():
        pltpu.sync_copy(acc.at[pl.ds(s * STRIPE, STRIPE)],
                        out_hbm.at[c].at[pl.ds(s * STRIPE, STRIPE)])

    @pl.when(s == NUM_TILES - 1)
    def _store_last():
        pltpu.sync_copy(
            acc.at[pl.ds((NUM_TILES - 1) * STRIPE, LAST_STRIPE)],
            out_hbm.at[c].at[pl.ds((NUM_TILES - 1) * STRIPE, LAST_STRIPE)])


@jax.jit
def _message_passing(x2, src_r, dst_r):
    mesh = plsc.VectorSubcoreMesh(core_axis_name="c", subcore_axis_name="s")
    run = pl.kernel(
        _sc_body,
        out_type=jax.ShapeDtypeStruct((2, N_NODES, D_HALF), jnp.float32),
        mesh=mesh,
        scratch_types=(
            [pltpu.VMEM((CHUNK, D_HALF), jnp.float32)] * NBUF +
            [pltpu.VMEM((2, CHUNK), jnp.int32)] * 8 +
            [pltpu.VMEM_SHARED((N_NODES, D_HALF), jnp.float32)] +
            [pltpu.SemaphoreType.DMA] * 12
        ),
        compiler_params=pltpu.CompilerParams(use_tc_tiling_on_sc=False),
    )
    return run(x2, src_r, dst_r)


def kernel(x, edge_index):
    x2 = jnp.moveaxis(x.reshape(N_NODES, 2, D_HALF), 1, 0)
    src_r = edge_index[0].reshape(NUM_TILES, CHUNKS_PER_TILE, 1, CHUNK)
    dst_r = edge_index[1].reshape(NUM_TILES, CHUNKS_PER_TILE, 1, CHUNK)
    out2 = _message_passing(x2, src_r, dst_r)
    return jnp.moveaxis(out2, 0, 1).reshape(N_NODES, D_FEAT)


# speedup vs baseline: 1.5380x; 1.0020x over previous
import jax
import jax.numpy as jnp
from jax import lax
from jax.experimental import pallas as pl
from jax.experimental.pallas import tpu as pltpu
from jax.experimental.pallas import tpu_sc as plsc

N_NODES = 10000
N_EDGES = 160000
D_FEAT = 256
D_HALF = D_FEAT // 2

NUM_TILES = 16
CHUNK = 80
CHUNKS_TOTAL = N_EDGES // CHUNK
CHUNKS_PER_TILE = CHUNKS_TOTAL // NUM_TILES
STRIPE = 632
LAST_STRIPE = N_NODES - (NUM_TILES - 1) * STRIPE
NBUF = 4
ROUNDS = 15
LAST = CHUNKS_PER_TILE - 1


def _clear_stripe(acc, rows0, base, nrows):
    nfull = nrows // CHUNK
    rest = nrows - nfull * CHUNK

    @pl.loop(0, nfull)
    def _clear(k):
        pltpu.sync_copy(rows0, acc.at[pl.ds(base + k * CHUNK, CHUNK)])

    pltpu.sync_copy(rows0.at[pl.ds(0, rest)],
                    acc.at[pl.ds(base + nfull * CHUNK, rest)])


def _sc_body(x_hbm, src_hbm, dst_hbm, out_hbm, *refs):
    rows = refs[0:4]
    stages = refs[4:12]
    acc = refs[12]
    semr = refs[13:17]
    semi = refs[17:25]
    c = lax.axis_index("c")
    s = lax.axis_index("s")

    def idx_issue(j, st):
        pltpu.async_copy(src_hbm.at[s].at[j], stages[st].at[pl.ds(0, 1)],
                         semi[st])
        pltpu.async_copy(dst_hbm.at[s].at[j], stages[st].at[pl.ds(1, 1)],
                         semi[st])

    def idx_wait(j, st):
        pltpu.make_async_copy(src_hbm.at[s].at[j],
                              stages[st].at[pl.ds(0, 1)], semi[st]).wait()
        pltpu.make_async_copy(dst_hbm.at[s].at[j],
                              stages[st].at[pl.ds(1, 1)], semi[st]).wait()

    def gather(j, b, st):
        return pltpu.async_copy(x_hbm.at[c].at[stages[st].at[0]],
                                rows[b], semr[b])

    def wait_gather(j, b, st):
        pltpu.make_async_copy(x_hbm.at[c].at[stages[st].at[0]],
                              rows[b], semr[b]).wait()

    def scatter(b, st):
        return pltpu.async_copy(rows[b], acc.at[stages[st].at[1]],
                                semr[b], add=True)

    for j in range(8):
        idx_issue(j, (j % 4) * 2 + j // 4)
    for j in range(1, 4):
        idx_wait(j, j * 2)
        gather(j, j, j * 2)

    @pl.loop(0, CHUNK)
    def _zero_rows(r):
        @pl.loop(0, D_HALF // 16)
        def _zero_lanes(j):
            rows[0][r, pl.ds(j * 16, 16)] = jnp.zeros((16,), jnp.float32)

    @pl.when(s < NUM_TILES - 1)
    def _clear_full():
        _clear_stripe(acc, rows[0], s * STRIPE, STRIPE)

    @pl.when(s == NUM_TILES - 1)
    def _clear_last():
        _clear_stripe(acc, rows[0], (NUM_TILES - 1) * STRIPE, LAST_STRIPE)

    idx_wait(0, 0)
    gather(0, 0, 0)

    plsc.subcore_barrier()

    sc = [None] * NBUF

    @pl.loop(0, ROUNDS)
    def _round(i):
        j8 = 8 * i
        for b in range(4):
            wait_gather(j8 + b, b, b * 2)
            sc[b] = scatter(b, b * 2)
        for b in range(4):
            sc[b].wait()
            idx_wait(j8 + 4 + b, b * 2 + 1)
            gather(j8 + 4 + b, b, b * 2 + 1)
            idx_issue(jnp.minimum(j8 + 8 + b, LAST), b * 2)
        for b in range(4):
            wait_gather(j8 + 4 + b, b, b * 2 + 1)
            sc[b] = scatter(b, b * 2 + 1)
        for b in range(4):
            sc[b].wait()
            idx_wait(jnp.minimum(j8 + 8 + b, LAST), b * 2)
            gather(jnp.minimum(j8 + 8 + b, LAST), b, b * 2)
            idx_issue(jnp.minimum(j8 + 12 + b, LAST), b * 2 + 1)

    for b in range(4):
        wait_gather(120 + b, b, b * 2)
        sc[b] = scatter(b, b * 2)
    sc[0].wait()
    idx_wait(LAST, 1)
    g = gather(LAST, 0, 1)
    for b in range(1, 4):
        sc[b].wait()
        idx_wait(LAST, b * 2 + 1)
    g.wait()
    pltpu.sync_copy(rows[0], acc.at[stages[1].at[1]], add=True)

    plsc.subcore_barrier()

    @pl.when(s < NUM_TILES - 1)
    def _store_full():
        pltpu.sync_copy(acc.at[pl.ds(s * STRIPE, STRIPE)],
                        out_hbm.at[c].at[pl.ds(s * STRIPE, STRIPE)])

    @pl.when(s == NUM_TILES - 1)
    def _store_last():
        pltpu.sync_copy(
            acc.at[pl.ds((NUM_TILES - 1) * STRIPE, LAST_STRIPE)],
            out_hbm.at[c].at[pl.ds((NUM_TILES - 1) * STRIPE, LAST_STRIPE)])


@jax.jit
def _message_passing(x2, src_r, dst_r):
    mesh = plsc.VectorSubcoreMesh(core_axis_name="c", subcore_axis_name="s")
    run = pl.kernel(
        _sc_body,
        out_type=jax.ShapeDtypeStruct((2, N_NODES, D_HALF), jnp.float32),
        mesh=mesh,
        scratch_types=(
            [pltpu.VMEM((CHUNK, D_HALF), jnp.float32)] * NBUF +
            [pltpu.VMEM((2, CHUNK), jnp.int32)] * 8 +
            [pltpu.VMEM_SHARED((N_NODES, D_HALF), jnp.float32)] +
            [pltpu.SemaphoreType.DMA] * 12
        ),
        compiler_params=pltpu.CompilerParams(use_tc_tiling_on_sc=False),
    )
    return run(x2, src_r, dst_r)


def kernel(x, edge_index):
    x2 = jnp.moveaxis(x.reshape(N_NODES, 2, D_HALF), 1, 0)
    src_r = edge_index[0].reshape(NUM_TILES, CHUNKS_PER_TILE, 1, CHUNK)
    dst_r = edge_index[1].reshape(NUM_TILES, CHUNKS_PER_TILE, 1, CHUNK)
    out2 = _message_passing(x2, src_r, dst_r)
    return jnp.moveaxis(out2, 0, 1).reshape(N_NODES, D_FEAT)
